# Initial kernel scaffold; baseline (speedup 1.0000x reference)
import jax
import jax.numpy as jnp
from jax.experimental import pallas as pl


def kernel(hiddens, entity_keys, encoded_sents, U, V, W, indices):
    raise NotImplementedError("write your pallas kernel here")


# speedup vs baseline: 1.5606x; 1.5606x over previous
import functools

import jax
import jax.numpy as jnp
from jax.experimental import pallas as pl
from jax.experimental.pallas import tpu as pltpu

R = 64


def _prep_body(es_ref, w_ref, u_ref, v_ref, sw_ref, uv_ref):
    sw_ref[...] = jnp.dot(es_ref[...], w_ref[...],
                          preferred_element_type=jnp.float32)
    uv_ref[...] = u_ref[...] + v_ref[...]


def _main_body(starts_ref, sidx_ref, h_ref, k_ref, es_ref, sw_ref, uv_ref,
               out_ref, huv_scr, *, blk_rows, ents, dim):
    b = pl.program_id(0)
    h = h_ref[...]
    out_ref[...] = h
    huv_scr[...] = jnp.dot(h.reshape(blk_rows * ents, dim), uv_ref[...],
                           preferred_element_type=jnp.float32)
    s0 = starts_ref[b]
    s1 = starts_ref[b + 1]

    def body(c, _):
        r = sidx_ref[c] - b * blk_rows
        es_c = es_ref[pl.ds(c, 1), :]
        hr = h_ref[r]
        kr = k_ref[r]
        g = jnp.sum((hr + kr) * es_c, axis=-1, keepdims=True)
        gate = jax.nn.sigmoid(g)
        ht = jnp.maximum(huv_scr[pl.ds(r * ents, ents), :]
                         + sw_ref[pl.ds(c, 1), :], 0.0)
        out_ref[r] = out_ref[r] + gate * ht
        return 0

    jax.lax.fori_loop(s0, s1, body, 0)

    o = out_ref[...]
    ssq = jnp.sum(o * o, axis=-1, keepdims=True)
    out_ref[...] = o * jax.lax.rsqrt(jnp.maximum(ssq, 1e-12))


def kernel(hiddens, entity_keys, encoded_sents, U, V, W, indices):
    B, E, D = hiddens.shape
    C = encoded_sents.shape[0]
    nb = B // R

    idx = indices.astype(jnp.int32)
    sidx, perm = jax.lax.sort_key_val(idx, jnp.arange(C, dtype=jnp.int32))
    es_sorted = jnp.take(encoded_sents, perm, axis=0)
    bounds = jnp.arange(nb + 1, dtype=jnp.int32) * R
    starts = jnp.searchsorted(sidx, bounds).astype(jnp.int32)

    sw, uv = pl.pallas_call(
        _prep_body,
        out_shape=(
            jax.ShapeDtypeStruct((C, D), jnp.float32),
            jax.ShapeDtypeStruct((D, D), jnp.float32),
        ),
    )(es_sorted, W, U, V)

    body = functools.partial(_main_body, blk_rows=R, ents=E, dim=D)
    grid_spec = pltpu.PrefetchScalarGridSpec(
        num_scalar_prefetch=2,
        grid=(nb,),
        in_specs=[
            pl.BlockSpec((R, E, D), lambda b, *_: (b, 0, 0)),
            pl.BlockSpec((R, E, D), lambda b, *_: (b, 0, 0)),
            pl.BlockSpec((C, D), lambda b, *_: (0, 0)),
            pl.BlockSpec((C, D), lambda b, *_: (0, 0)),
            pl.BlockSpec((D, D), lambda b, *_: (0, 0)),
        ],
        out_specs=pl.BlockSpec((R, E, D), lambda b, *_: (b, 0, 0)),
        scratch_shapes=[pltpu.VMEM((R * E, D), jnp.float32)],
    )
    out = pl.pallas_call(
        body,
        grid_spec=grid_spec,
        out_shape=jax.ShapeDtypeStruct((B, E, D), jnp.float32),
        compiler_params=pltpu.CompilerParams(
            dimension_semantics=("arbitrary",)),
    )(starts, sidx, hiddens, entity_keys, es_sorted, sw, uv)
    return out


# speedup vs baseline: 1.5611x; 1.0004x over previous
import functools

import jax
import jax.numpy as jnp
from jax.experimental import pallas as pl
from jax.experimental.pallas import tpu as pltpu

R = 64


def _prep_body(es_ref, w_ref, u_ref, v_ref, sw_ref, uv_ref):
    sw_ref[...] = jnp.dot(es_ref[...], w_ref[...],
                          preferred_element_type=jnp.float32)
    uv_ref[...] = u_ref[...] + v_ref[...]


def _main_body(starts_ref, sidx_ref, h_ref, k_ref, es_ref, sw_ref, uv_ref,
               out_ref, huv_scr, *, blk_rows, ents, dim):
    b = pl.program_id(0)
    h = h_ref[...]
    out_ref[...] = h
    huv_scr[...] = jnp.dot(h.reshape(blk_rows * ents, dim), uv_ref[...],
                           preferred_element_type=jnp.float32)
    s0 = starts_ref[b]
    s1 = starts_ref[b + 1]

    def body(c, _):
        r = sidx_ref[c] - b * blk_rows
        es_c = es_ref[pl.ds(c, 1), :]
        hr = h_ref[r]
        kr = k_ref[r]
        g = jnp.sum((hr + kr) * es_c, axis=-1, keepdims=True)
        gate = jax.nn.sigmoid(g)
        ht = jnp.maximum(huv_scr[pl.ds(r * ents, ents), :]
                         + sw_ref[pl.ds(c, 1), :], 0.0)
        out_ref[r] = out_ref[r] + gate * ht
        return 0

    jax.lax.fori_loop(s0, s1, body, 0)

    o = out_ref[...]
    ssq = jnp.sum(o * o, axis=-1, keepdims=True)
    out_ref[...] = o * jax.lax.rsqrt(jnp.maximum(ssq, 1e-12))


def kernel(hiddens, entity_keys, encoded_sents, U, V, W, indices):
    B, E, D = hiddens.shape
    C = encoded_sents.shape[0]
    nb = B // R

    idx = indices.astype(jnp.int32)
    sidx, perm = jax.lax.sort_key_val(idx, jnp.arange(C, dtype=jnp.int32))
    es_sorted = jnp.take(encoded_sents, perm, axis=0)
    bounds = jnp.arange(nb + 1, dtype=jnp.int32) * R
    starts = jnp.searchsorted(sidx, bounds).astype(jnp.int32)

    sw, uv = pl.pallas_call(
        _prep_body,
        out_shape=(
            jax.ShapeDtypeStruct((C, D), jnp.float32),
            jax.ShapeDtypeStruct((D, D), jnp.float32),
        ),
    )(es_sorted, W, U, V)

    body = functools.partial(_main_body, blk_rows=R, ents=E, dim=D)
    grid_spec = pltpu.PrefetchScalarGridSpec(
        num_scalar_prefetch=2,
        grid=(nb,),
        in_specs=[
            pl.BlockSpec((R, E, D), lambda b, *_: (b, 0, 0)),
            pl.BlockSpec((R, E, D), lambda b, *_: (b, 0, 0)),
            pl.BlockSpec((C, D), lambda b, *_: (0, 0)),
            pl.BlockSpec((C, D), lambda b, *_: (0, 0)),
            pl.BlockSpec((D, D), lambda b, *_: (0, 0)),
        ],
        out_specs=pl.BlockSpec((R, E, D), lambda b, *_: (b, 0, 0)),
        scratch_shapes=[pltpu.VMEM((R * E, D), jnp.float32)],
    )
    out = pl.pallas_call(
        body,
        grid_spec=grid_spec,
        out_shape=jax.ShapeDtypeStruct((B, E, D), jnp.float32),
        compiler_params=pltpu.CompilerParams(
            dimension_semantics=("parallel",)),
    )(starts, sidx, hiddens, entity_keys, es_sorted, sw, uv)
    return out
